# Initial kernel scaffold; baseline (speedup 1.0000x reference)
import jax
import jax.numpy as jnp
from jax.experimental import pallas as pl


def kernel(x, edge_index, emb, w_gate, W, b, gamma, beta):
    raise NotImplementedError("write your pallas kernel here")


# speedup vs baseline: 32.0919x; 32.0919x over previous
import functools

import jax
import jax.numpy as jnp
from jax import lax
from jax.experimental import pallas as pl
from jax.experimental.pallas import tpu as pltpu, tpu_sc as plsc

N = 10000
E = 160000
D = 256
NE = 8
VOCAB = 129
NLAYER = 4
EPS = 1e-5

NC, NS = 2, 16
ET = E // NS
KC = 80
NCHUNK = ET // KC
RT = N // NS
NPAD = 10240

_f32 = jnp.float32


@functools.cache
def _sc_mesh():
    return plsc.VectorSubcoreMesh(core_axis_name="c", subcore_axis_name="s",
                                  num_cores=NC, num_subcores=NS)


_NSEM = 6


def _deg_body(src2_hbm, dst2_hbm, outs_hbm, outd_hbm, ones_rows, iall, zrow, acc, ssem):
    c = lax.axis_index("c")
    s = lax.axis_index("s")
    zeros16 = jnp.zeros((16,), _f32)
    e0 = jnp.where(lax.iota(jnp.int32, 16) == 0, 1.0, 0.0).astype(_f32)

    def fill(i, _):
        ones_rows[i] = e0
        return _
    lax.fori_loop(0, KC, fill, 0)

    def zfill(i, _):
        zrow[i] = zeros16
        return _
    lax.fori_loop(0, 40, zfill, 0)

    def zacc(i, _):
        pltpu.sync_copy(zrow, acc.at[pl.ds(s * 640 + i * 40, 40)])
        return _
    lax.fori_loop(0, 16, zacc, 0)

    @pl.when(c == 0)
    def _():
        pltpu.sync_copy(src2_hbm.at[s], iall)

    @pl.when(c == 1)
    def _():
        pltpu.sync_copy(dst2_hbm.at[s], iall)

    plsc.subcore_barrier()

    def step(j, _):
        pltpu.sync_copy(ones_rows, acc.at[iall.at[j]], add=True)
        return _
    lax.fori_loop(0, NCHUNK, step, 0)

    plsc.subcore_barrier()

    @pl.when(c == 0)
    def _():
        pltpu.sync_copy(acc.at[pl.ds(s * 640, 640)],
                        outs_hbm.at[pl.ds(s * 640, 640)])

    @pl.when(c == 1)
    def _():
        pltpu.sync_copy(acc.at[pl.ds(s * 640, 640)],
                        outd_hbm.at[pl.ds(s * 640, 640)])


@functools.cache
def _deg_call():
    return pl.kernel(
        _deg_body,
        out_type=[jax.ShapeDtypeStruct((NPAD, 16), _f32),
                  jax.ShapeDtypeStruct((NPAD, 16), _f32)],
        mesh=_sc_mesh(),
        scratch_types=[
            pltpu.VMEM((KC, 16), _f32),
            pltpu.VMEM((NCHUNK, KC), jnp.int32),
            pltpu.VMEM((40, 16), _f32),
            pltpu.VMEM_SHARED((NPAD, 16), _f32),
            pltpu.SemaphoreType.DMA,
        ],
    )


_DELTA = 1
_NRING = 2


def _seg_body(hn0_hbm, hn1_hbm, pidx_hbm, out0_hbm, out1_hbm,
              pall, ibufA, dbufA, ibufB, dbufB, rows, acc, gsem, gsem2, ssem):
    c = lax.axis_index("c")
    s = lax.axis_index("s")
    zeros16 = jnp.zeros((16,), _f32)

    def zfill(i, _):
        rows[0, i // 8, pl.ds((i % 8) * 16, 16)] = zeros16
        return _
    lax.fori_loop(0, KC * 8, zfill, 0)

    def zacc(i, _):
        pltpu.sync_copy(rows.at[0], acc.at[pl.ds(s * RT + i * KC, KC)])
        return _
    lax.fori_loop(0, RT // KC, zacc, 0)
    pltpu.sync_copy(rows.at[0, pl.ds(0, RT % KC)],
                    acc.at[pl.ds(s * RT + (RT // KC) * KC, RT % KC)])

    pltpu.sync_copy(pidx_hbm.at[s], pall)
    plsc.subcore_barrier()

    def unpack(j, ibuf, dbuf):
        def u(t, _):
            v = pall[j, pl.ds(t * 16, 16)]
            ibuf[pl.ds(t * 16, 16)] = lax.shift_right_logical(v, 14)
            dbuf[pl.ds(t * 16, 16)] = jnp.bitwise_and(v, 16383)
            return _
        lax.fori_loop(0, KC // 16, u, 0)

    def pipeline(hn_hbm):
        def chunk(j, _):
            unpack(j, ibufA, dbufA)
            pltpu.async_copy(hn_hbm.at[ibufA], rows.at[0], gsem).wait()
            pltpu.sync_copy(rows.at[0], acc.at[dbufA], add=True)
            return _
        lax.fori_loop(0, NCHUNK, chunk, 0)

    @pl.when(c == 0)
    def _p0():
        pipeline(hn0_hbm)

    @pl.when(c == 1)
    def _p1():
        pipeline(hn1_hbm)

    plsc.subcore_barrier()
    nlast = N - (NS - 1) * 624

    @pl.when(jnp.logical_and(c == 0, s < NS - 1))
    def _():
        pltpu.sync_copy(acc.at[pl.ds(s * 624, 624)],
                        out0_hbm.at[pl.ds(s * 624, 624)])

    @pl.when(jnp.logical_and(c == 0, s == NS - 1))
    def _():
        pltpu.sync_copy(acc.at[pl.ds((NS - 1) * 624, nlast)],
                        out0_hbm.at[pl.ds((NS - 1) * 624, nlast)])

    @pl.when(jnp.logical_and(c == 1, s < NS - 1))
    def _():
        pltpu.sync_copy(acc.at[pl.ds(s * 624, 624)],
                        out1_hbm.at[pl.ds(s * 624, 624)])

    @pl.when(jnp.logical_and(c == 1, s == NS - 1))
    def _():
        pltpu.sync_copy(acc.at[pl.ds((NS - 1) * 624, nlast)],
                        out1_hbm.at[pl.ds((NS - 1) * 624, nlast)])


@functools.cache
def _seg_call():
    return pl.kernel(
        _seg_body,
        out_type=[jax.ShapeDtypeStruct((N, 128), _f32),
                  jax.ShapeDtypeStruct((N, 128), _f32)],
        mesh=_sc_mesh(),
        scratch_types=[
            pltpu.VMEM((NCHUNK, KC), jnp.int32),
            pltpu.VMEM((KC,), jnp.int32),
            pltpu.VMEM((KC,), jnp.int32),
            pltpu.VMEM((KC,), jnp.int32),
            pltpu.VMEM((KC,), jnp.int32),
            pltpu.VMEM((2, KC, 128), _f32),
            pltpu.VMEM_SHARED((N, 128), _f32),
            pltpu.SemaphoreType.DMA,
            pltpu.SemaphoreType.DMA,
            pltpu.SemaphoreType.DMA,
        ],
    )


BR = 1000
GRID = N // BR


def _gates_from(h, wg):
    logits = jnp.dot(h, wg, preferred_element_type=_f32)
    m1 = jnp.max(logits, axis=-1, keepdims=True)
    idx = lax.broadcasted_iota(jnp.int32, logits.shape, 1)
    ismax = logits >= m1
    first = jnp.min(jnp.where(ismax, idx, NE), axis=-1, keepdims=True)
    masked = jnp.where(idx == first, jnp.float32(-3e38), logits)
    m2 = jnp.max(masked, axis=-1, keepdims=True)
    top = logits >= m2
    eg = jnp.exp(logits - m1)
    denom = 1.0 + jnp.exp(m2 - m1)
    return jnp.where(top, eg / denom, jnp.float32(0.0))


def _pre0_body(x_ref, od_ref, id_ref, emb_ref, wg_ref,
               h_ref, g_ref, hn_ref, sn_ref, dn_ref):
    xb = x_ref[...]
    oh = (xb == lax.broadcasted_iota(jnp.int32, (BR, VOCAB), 1)).astype(_f32)
    h = jnp.dot(oh, emb_ref[...], preferred_element_type=_f32)
    sn = lax.rsqrt(jnp.maximum(od_ref[...], 1.0))
    dn = lax.rsqrt(jnp.maximum(id_ref[...], 1.0))
    h_ref[...] = h
    sn_ref[...] = sn
    dn_ref[...] = dn
    hn_ref[...] = h * sn
    g_ref[...] = _gates_from(h, wg_ref[...])


_pre0_call = pl.pallas_call(
    _pre0_body,
    grid=(GRID,),
    in_specs=[
        pl.BlockSpec((BR, 1), lambda i: (i, 0)),
        pl.BlockSpec((BR, 1), lambda i: (i, 0)),
        pl.BlockSpec((BR, 1), lambda i: (i, 0)),
        pl.BlockSpec((VOCAB, D), lambda i: (0, 0)),
        pl.BlockSpec((D, NE), lambda i: (0, 0)),
    ],
    out_specs=[
        pl.BlockSpec((BR, D), lambda i: (i, 0)),
        pl.BlockSpec((BR, NE), lambda i: (i, 0)),
        pl.BlockSpec((BR, D), lambda i: (i, 0)),
        pl.BlockSpec((BR, 1), lambda i: (i, 0)),
        pl.BlockSpec((BR, 1), lambda i: (i, 0)),
    ],
    out_shape=[
        jax.ShapeDtypeStruct((N, D), _f32),
        jax.ShapeDtypeStruct((N, NE), _f32),
        jax.ShapeDtypeStruct((N, D), _f32),
        jax.ShapeDtypeStruct((N, 1), _f32),
        jax.ShapeDtypeStruct((N, 1), _f32),
    ],
)


GRG = 2
BRG = N // GRG


def _gram_prep_body(a0_ref, a1_ref, dn_ref, w_ref, ga_ref,
                    s_ref, wp_ref, c_ref):
    i = pl.program_id(0)
    dn = dn_ref[...]
    ad0 = a0_ref[...] * dn
    ad1 = a1_ref[...] * dn

    @pl.when(i == 0)
    def _():
        c_ref[...] = jnp.zeros_like(c_ref)
        s_ref[...] = jnp.zeros_like(s_ref)

    dim = (((0,), (0,)), ((), ()))
    c_ref[0:128, 0:128] += lax.dot_general(ad0, ad0, dim,
                                           preferred_element_type=_f32)
    c_ref[0:128, 128:256] += lax.dot_general(ad0, ad1, dim,
                                             preferred_element_type=_f32)
    c_ref[128:256, 0:128] += lax.dot_general(ad1, ad0, dim,
                                             preferred_element_type=_f32)
    c_ref[128:256, 128:256] += lax.dot_general(ad1, ad1, dim,
                                               preferred_element_type=_f32)
    s_ref[:, 0:128] += jnp.sum(ad0, axis=0, keepdims=True)
    s_ref[:, 128:256] += jnp.sum(ad1, axis=0, keepdims=True)

    @pl.when(i == GRG - 1)
    def _():
        sbar = s_ref[...] * (1.0 / N)
        outer = lax.dot_general(sbar, sbar, dim, preferred_element_type=_f32)
        cc = c_ref[...] * (1.0 / N) - outer
        for e in range(NE):
            w = w_ref[e]
            cw = jnp.dot(cc, w, preferred_element_type=_f32)
            var = jnp.sum(cw * w, axis=0, keepdims=True)
            a = ga_ref[e] * lax.rsqrt(var + EPS)
            wp_ref[:, e * D:(e + 1) * D] = w * a


_gram_prep_call = pl.pallas_call(
    _gram_prep_body,
    grid=(GRG,),
    in_specs=[
        pl.BlockSpec((BRG, 128), lambda i: (i, 0)),
        pl.BlockSpec((BRG, 128), lambda i: (i, 0)),
        pl.BlockSpec((BRG, 1), lambda i: (i, 0)),
        pl.BlockSpec((NE, D, D), lambda i: (0, 0, 0)),
        pl.BlockSpec((NE, 1, D), lambda i: (0, 0, 0)),
    ],
    out_specs=[
        pl.BlockSpec((1, D), lambda i: (0, 0)),
        pl.BlockSpec((D, NE * D), lambda i: (0, 0)),
    ],
    out_shape=[
        jax.ShapeDtypeStruct((1, D), _f32),
        jax.ShapeDtypeStruct((D, NE * D), _f32),
    ],
    scratch_shapes=[pltpu.VMEM((D, D), _f32)],
)


def _combine_body(with_next, a0_ref, a1_ref, dn_ref, s_ref, wp_ref, g_ref,
                  beta_ref, wgn_ref, sn_ref, h_ref, gn_ref, hn_ref):
    sbar = s_ref[...] * (1.0 / N)
    dn = dn_ref[...]
    adc0 = a0_ref[...] * dn - sbar[:, 0:128]
    adc1 = a1_ref[...] * dn - sbar[:, 128:256]
    y_all = (jnp.dot(adc0, wp_ref[0:128, :], preferred_element_type=_f32)
             + jnp.dot(adc1, wp_ref[128:256, :], preferred_element_type=_f32))
    g = g_ref[...]
    y = jnp.dot(g, beta_ref[...], preferred_element_type=_f32)
    for e in range(NE):
        y = y + g[:, e:e + 1] * y_all[:, e * D:(e + 1) * D]
    h = y + jnp.maximum(y, 0.0)
    h_ref[...] = h
    if with_next:
        gn_ref[...] = _gates_from(h, wgn_ref[...])
        hn_ref[...] = h * sn_ref[...]


_combine_next_call = pl.pallas_call(
    functools.partial(_combine_body, True),
    grid=(GRID,),
    in_specs=[
        pl.BlockSpec((BR, 128), lambda i: (i, 0)),
        pl.BlockSpec((BR, 128), lambda i: (i, 0)),
        pl.BlockSpec((BR, 1), lambda i: (i, 0)),
        pl.BlockSpec((1, D), lambda i: (0, 0)),
        pl.BlockSpec((D, NE * D), lambda i: (0, 0)),
        pl.BlockSpec((BR, NE), lambda i: (i, 0)),
        pl.BlockSpec((NE, D), lambda i: (0, 0)),
        pl.BlockSpec((D, NE), lambda i: (0, 0)),
        pl.BlockSpec((BR, 1), lambda i: (i, 0)),
    ],
    out_specs=[
        pl.BlockSpec((BR, D), lambda i: (i, 0)),
        pl.BlockSpec((BR, NE), lambda i: (i, 0)),
        pl.BlockSpec((BR, D), lambda i: (i, 0)),
    ],
    out_shape=[
        jax.ShapeDtypeStruct((N, D), _f32),
        jax.ShapeDtypeStruct((N, NE), _f32),
        jax.ShapeDtypeStruct((N, D), _f32),
    ],
)


def _combine_last_body(a0_ref, a1_ref, dn_ref, s_ref, wp_ref, g_ref,
                       beta_ref, h_ref):
    _combine_body(False, a0_ref, a1_ref, dn_ref, s_ref, wp_ref, g_ref,
                  beta_ref, None, None, h_ref, None, None)


_combine_last_call = pl.pallas_call(
    _combine_last_body,
    grid=(GRID,),
    in_specs=[
        pl.BlockSpec((BR, 128), lambda i: (i, 0)),
        pl.BlockSpec((BR, 128), lambda i: (i, 0)),
        pl.BlockSpec((BR, 1), lambda i: (i, 0)),
        pl.BlockSpec((1, D), lambda i: (0, 0)),
        pl.BlockSpec((D, NE * D), lambda i: (0, 0)),
        pl.BlockSpec((BR, NE), lambda i: (i, 0)),
        pl.BlockSpec((NE, D), lambda i: (0, 0)),
    ],
    out_specs=pl.BlockSpec((BR, D), lambda i: (i, 0)),
    out_shape=jax.ShapeDtypeStruct((N, D), _f32),
)


def kernel(x, edge_index, emb, w_gate, W, b, gamma, beta):
    del b
    src2 = edge_index[0].astype(jnp.int32).reshape(NS, NCHUNK, KC)
    dst2 = edge_index[1].astype(jnp.int32).reshape(NS, NCHUNK, KC)
    pidx = src2 * 16384 + dst2

    degs, degd = _deg_call()(src2, dst2)
    out_deg = degs[:N, 0].reshape(N, 1)
    in_deg = degd[:N, 0].reshape(N, 1)

    x2 = x.astype(jnp.int32).reshape(N, 1)
    h, gates, hn, sn, dn = _pre0_call(x2, out_deg, in_deg, emb, w_gate[0])

    for l in range(NLAYER):
        a0, a1 = _seg_call()(hn[:, :128], hn[:, 128:], pidx)
        ssum, wp = _gram_prep_call(a0, a1, dn, W[l], gamma[l].reshape(NE, 1, D))
        if l < NLAYER - 1:
            h, gates, hn = _combine_next_call(a0, a1, dn, ssum, wp, gates,
                                              beta[l], w_gate[l + 1], sn)
        else:
            h = _combine_last_call(a0, a1, dn, ssum, wp, gates, beta[l])
    return h
